# baseline (speedup 1.0000x reference)
import jax
import jax.numpy as jnp
from jax import lax
from jax.experimental import pallas as pl


_BLOCK = 16000


def _phi_body(src_ref, et_ref, tgt_ref, out_ref):
    de = et_ref.shape[0]
    d = src_ref.shape[1]
    ones = jnp.full((de, d), 1.0 / de, jnp.float32)
    s = lax.dot_general(
        et_ref[...], ones, (((0,), (0,)), ((), ())),
        preferred_element_type=jnp.float32,
    )
    gate = jax.nn.sigmoid(s)
    out_ref[...] = src_ref[...] * gate + tgt_ref[...]


def kernel(src, e, tgt):
    n, d = src.shape
    de = e.shape[1]
    grid = n // _BLOCK
    return pl.pallas_call(
        _phi_body,
        grid=(grid,),
        in_specs=[
            pl.BlockSpec((_BLOCK, d), lambda i: (i, 0)),
            pl.BlockSpec((de, _BLOCK), lambda i: (0, i)),
            pl.BlockSpec((_BLOCK, d), lambda i: (i, 0)),
        ],
        out_specs=pl.BlockSpec((_BLOCK, d), lambda i: (i, 0)),
        out_shape=jax.ShapeDtypeStruct((n, d), src.dtype),
    )(src, e.T, tgt)


# speedup vs baseline: 1.0016x; 1.0016x over previous
import jax
import jax.numpy as jnp
from jax import lax
from jax.experimental import pallas as pl


_BLOCK = 12800


def _phi_body(src_ref, et_ref, tgt_ref, out_ref):
    de = et_ref.shape[0]
    d = src_ref.shape[1]
    ones = jnp.full((de, d), 1.0 / de, jnp.float32)
    s = lax.dot_general(
        et_ref[...], ones, (((0,), (0,)), ((), ())),
        preferred_element_type=jnp.float32,
    )
    gate = jax.nn.sigmoid(s)
    out_ref[...] = src_ref[...] * gate + tgt_ref[...]


def kernel(src, e, tgt):
    n, d = src.shape
    de = e.shape[1]
    grid = n // _BLOCK
    return pl.pallas_call(
        _phi_body,
        grid=(grid,),
        in_specs=[
            pl.BlockSpec((_BLOCK, d), lambda i: (i, 0)),
            pl.BlockSpec((de, _BLOCK), lambda i: (0, i)),
            pl.BlockSpec((_BLOCK, d), lambda i: (i, 0)),
        ],
        out_specs=pl.BlockSpec((_BLOCK, d), lambda i: (i, 0)),
        out_shape=jax.ShapeDtypeStruct((n, d), src.dtype),
    )(src, e.T, tgt)
